# baseline (speedup 1.0000x reference)
import functools

import jax
import jax.numpy as jnp
from jax import lax
from jax.experimental import pallas as pl
from jax.experimental.pallas import tpu as pltpu
from jax.experimental.pallas import tpu_sc as plsc

VOCAB = 32000
EMB = 512
TOTAL = 16384

NC = 2
NS = 16
NW = NC * NS
TPW = TOTAL // NW
CHUNK = 64
NCHUNK = TPW // CHUNK
NBUF = 3


def _lookup_kernel(idx_hbm, table_hbm, out_hbm, idx_v, *scratch):
    rows = scratch[:NBUF]
    gsem = scratch[NBUF:2 * NBUF]
    ssem = scratch[2 * NBUF:3 * NBUF]
    wid = lax.axis_index("s") * NC + lax.axis_index("c")
    base = wid * TPW
    pltpu.sync_copy(idx_hbm.at[wid], idx_v)

    def gather(j):
        b = j % NBUF
        return pltpu.async_copy(table_hbm.at[idx_v.at[j]], rows[b], gsem[b])

    def store(j):
        b = j % NBUF
        return pltpu.async_copy(
            rows[b], out_hbm.at[pl.ds(base + j * CHUNK, CHUNK)], ssem[b])

    gh = [None] * NCHUNK
    sh = [None] * NCHUNK
    for j in range(NBUF):
        gh[j] = gather(j)
    for j in range(NCHUNK):
        gh[j].wait()
        sh[j] = store(j)
        nj = j - 1 + NBUF
        if j >= 1 and nj < NCHUNK:
            sh[j - 1].wait()
            gh[nj] = gather(nj)
    for j in range(max(0, NCHUNK - NBUF), NCHUNK):
        sh[j].wait()


@jax.jit
def _lookup(flat_tokens, emb_table):
    idx = flat_tokens.reshape(NW, NCHUNK, CHUNK)
    mesh = plsc.VectorSubcoreMesh(core_axis_name="c", subcore_axis_name="s")
    run = pl.kernel(
        _lookup_kernel,
        mesh=mesh,
        out_type=jax.ShapeDtypeStruct((TOTAL, EMB), jnp.float32),
        scratch_types=(
            [pltpu.VMEM((NCHUNK, CHUNK), jnp.int32)]
            + [pltpu.VMEM((CHUNK, EMB), jnp.float32) for _ in range(NBUF)]
            + [pltpu.SemaphoreType.DMA for _ in range(2 * NBUF)]
        ),
    )
    return run(idx, emb_table)


def kernel(flat_tokens, cu_seqlens, emb_table):
    all_embs = _lookup(flat_tokens, emb_table)
    return (all_embs, cu_seqlens)


# speedup vs baseline: 1.0009x; 1.0009x over previous
import functools

import jax
import jax.numpy as jnp
from jax import lax
from jax.experimental import pallas as pl
from jax.experimental.pallas import tpu as pltpu
from jax.experimental.pallas import tpu_sc as plsc

VOCAB = 32000
EMB = 512
TOTAL = 16384

NC = 2
NS = 16
NW = NC * NS
TPW = TOTAL // NW
CHUNK = 64
NCHUNK = TPW // CHUNK
NBUF = 3


def _lookup_kernel(idx_hbm, table_hbm, out_hbm, idx_v, *scratch):
    rows = scratch[:NBUF]
    gsem = scratch[NBUF:2 * NBUF]
    ssem = scratch[2 * NBUF:3 * NBUF]
    isem = scratch[3 * NBUF]
    wid = lax.axis_index("s") * NC + lax.axis_index("c")
    base = wid * TPW
    ih = [pltpu.async_copy(idx_hbm.at[pl.ds(base + j * CHUNK, CHUNK)],
                           idx_v.at[j], isem)
          for j in range(NCHUNK)]

    def gather(j):
        b = j % NBUF
        return pltpu.async_copy(table_hbm.at[idx_v.at[j]], rows[b], gsem[b])

    def store(j):
        b = j % NBUF
        return pltpu.async_copy(
            rows[b], out_hbm.at[pl.ds(base + j * CHUNK, CHUNK)], ssem[b])

    for h in ih:
        h.wait()
    gh = [None] * NCHUNK
    sh = [None] * NCHUNK
    for j in range(NBUF):
        gh[j] = gather(j)
    for j in range(NCHUNK):
        gh[j].wait()
        sh[j] = store(j)
        nj = j - 1 + NBUF
        if j >= 1 and nj < NCHUNK:
            sh[j - 1].wait()
            gh[nj] = gather(nj)
    for j in range(max(0, NCHUNK - NBUF), NCHUNK):
        sh[j].wait()


@jax.jit
def _lookup(flat_tokens, emb_table):
    mesh = plsc.VectorSubcoreMesh(core_axis_name="c", subcore_axis_name="s")
    run = pl.kernel(
        _lookup_kernel,
        mesh=mesh,
        out_type=jax.ShapeDtypeStruct((TOTAL, EMB), jnp.float32),
        scratch_types=(
            [pltpu.VMEM((NCHUNK, CHUNK), jnp.int32)]
            + [pltpu.VMEM((CHUNK, EMB), jnp.float32) for _ in range(NBUF)]
            + [pltpu.SemaphoreType.DMA for _ in range(2 * NBUF + 1)]
        ),
    )
    return run(flat_tokens, emb_table)


def kernel(flat_tokens, cu_seqlens, emb_table):
    all_embs = _lookup(flat_tokens, emb_table)
    return (all_embs, cu_seqlens)


# speedup vs baseline: 1.0013x; 1.0004x over previous
import functools

import jax
import jax.numpy as jnp
from jax import lax
from jax.experimental import pallas as pl
from jax.experimental.pallas import tpu as pltpu
from jax.experimental.pallas import tpu_sc as plsc

VOCAB = 32000
EMB = 512
TOTAL = 16384

NC = 2
NS = 16
NW = NC * NS
TPW = TOTAL // NW
CHUNK = 64
NCHUNK = TPW // CHUNK
NBUF = 3


def _lookup_kernel(idx_hbm, table_hbm, out_hbm, idx_v, *scratch):
    rows = scratch[:NBUF]
    gsem = scratch[NBUF:2 * NBUF]
    ssem = scratch[2 * NBUF:3 * NBUF]
    isem = scratch[3 * NBUF]
    wid = lax.axis_index("s") * NC + lax.axis_index("c")
    base = wid * TPW
    ih = pltpu.async_copy(idx_hbm.at[pl.ds(base, TPW)], idx_v, isem)

    def gather(j):
        b = j % NBUF
        return pltpu.async_copy(
            table_hbm.at[idx_v.at[pl.ds(j * CHUNK, CHUNK)]], rows[b], gsem[b])

    def store(j):
        b = j % NBUF
        return pltpu.async_copy(
            rows[b], out_hbm.at[pl.ds(base + j * CHUNK, CHUNK)], ssem[b])

    ih.wait()
    gh = [None] * NCHUNK
    sh = [None] * NCHUNK
    for j in range(NBUF):
        gh[j] = gather(j)
    for j in range(NCHUNK):
        gh[j].wait()
        sh[j] = store(j)
        nj = j - 1 + NBUF
        if j >= 1 and nj < NCHUNK:
            sh[j - 1].wait()
            gh[nj] = gather(nj)
    for j in range(max(0, NCHUNK - NBUF), NCHUNK):
        sh[j].wait()


@jax.jit
def _lookup(flat_tokens, emb_table):
    mesh = plsc.VectorSubcoreMesh(core_axis_name="c", subcore_axis_name="s")
    run = pl.kernel(
        _lookup_kernel,
        mesh=mesh,
        out_type=jax.ShapeDtypeStruct((TOTAL, EMB), jnp.float32),
        scratch_types=(
            [pltpu.VMEM((TPW,), jnp.int32)]
            + [pltpu.VMEM((CHUNK, EMB), jnp.float32) for _ in range(NBUF)]
            + [pltpu.SemaphoreType.DMA for _ in range(2 * NBUF + 1)]
        ),
    )
    return run(flat_tokens, emb_table)


def kernel(flat_tokens, cu_seqlens, emb_table):
    all_embs = _lookup(flat_tokens, emb_table)
    return (all_embs, cu_seqlens)


# speedup vs baseline: 1.0109x; 1.0095x over previous
import functools

import jax
import jax.numpy as jnp
from jax import lax
from jax.experimental import pallas as pl
from jax.experimental.pallas import tpu as pltpu
from jax.experimental.pallas import tpu_sc as plsc

VOCAB = 32000
EMB = 512
TOTAL = 16384

NC = 2
NS = 16
NW = NC * NS
TPW = TOTAL // NW
CHUNK = 32
NCHUNK = TPW // CHUNK
NBUF = 4


def _lookup_kernel(idx_hbm, table_hbm, out_hbm, idx_v, *scratch):
    rows = scratch[:NBUF]
    gsem = scratch[NBUF:2 * NBUF]
    ssem = scratch[2 * NBUF:3 * NBUF]
    isem = scratch[3 * NBUF]
    wid = lax.axis_index("s") * NC + lax.axis_index("c")
    base = wid * TPW
    ih = pltpu.async_copy(idx_hbm.at[pl.ds(base, TPW)], idx_v, isem)

    def gather(j):
        b = j % NBUF
        return pltpu.async_copy(
            table_hbm.at[idx_v.at[pl.ds(j * CHUNK, CHUNK)]], rows[b], gsem[b])

    def store(j):
        b = j % NBUF
        return pltpu.async_copy(
            rows[b], out_hbm.at[pl.ds(base + j * CHUNK, CHUNK)], ssem[b])

    ih.wait()
    gh = [None] * NCHUNK
    sh = [None] * NCHUNK
    for j in range(NBUF):
        gh[j] = gather(j)
    for j in range(NCHUNK):
        gh[j].wait()
        sh[j] = store(j)
        nj = j - 1 + NBUF
        if j >= 1 and nj < NCHUNK:
            sh[j - 1].wait()
            gh[nj] = gather(nj)
    for j in range(max(0, NCHUNK - NBUF), NCHUNK):
        sh[j].wait()


@jax.jit
def _lookup(flat_tokens, emb_table):
    mesh = plsc.VectorSubcoreMesh(core_axis_name="c", subcore_axis_name="s")
    run = pl.kernel(
        _lookup_kernel,
        mesh=mesh,
        out_type=jax.ShapeDtypeStruct((TOTAL, EMB), jnp.float32),
        scratch_types=(
            [pltpu.VMEM((TPW,), jnp.int32)]
            + [pltpu.VMEM((CHUNK, EMB), jnp.float32) for _ in range(NBUF)]
            + [pltpu.SemaphoreType.DMA for _ in range(2 * NBUF + 1)]
        ),
    )
    return run(flat_tokens, emb_table)


def kernel(flat_tokens, cu_seqlens, emb_table):
    all_embs = _lookup(flat_tokens, emb_table)
    return (all_embs, cu_seqlens)


# speedup vs baseline: 1.0276x; 1.0165x over previous
import functools

import jax
import jax.numpy as jnp
from jax import lax
from jax.experimental import pallas as pl
from jax.experimental.pallas import tpu as pltpu
from jax.experimental.pallas import tpu_sc as plsc

VOCAB = 32000
EMB = 512
TOTAL = 16384

NC = 2
NS = 16
NW = NC * NS
TPW = TOTAL // NW
CHUNK = 32
NCHUNK = TPW // CHUNK
NBUF = 4


def _lookup_kernel(idx_hbm, cu_hbm, table_hbm, out_hbm, bnd_hbm,
                   idx_v, bnd_v, *scratch):
    rows = scratch[:NBUF]
    gsem = scratch[NBUF:2 * NBUF]
    ssem = scratch[2 * NBUF:3 * NBUF]
    isem = scratch[3 * NBUF]
    wid = lax.axis_index("s") * NC + lax.axis_index("c")
    base = wid * TPW
    ih = pltpu.async_copy(idx_hbm.at[pl.ds(base, TPW)], idx_v, isem)

    @pl.when(wid == 0)
    def _():
        pltpu.sync_copy(cu_hbm, bnd_v)
        pltpu.sync_copy(bnd_v, bnd_hbm)

    def gather(j):
        b = j % NBUF
        return pltpu.async_copy(
            table_hbm.at[idx_v.at[pl.ds(j * CHUNK, CHUNK)]], rows[b], gsem[b])

    def store(j):
        b = j % NBUF
        return pltpu.async_copy(
            rows[b], out_hbm.at[pl.ds(base + j * CHUNK, CHUNK)], ssem[b])

    ih.wait()
    gh = [None] * NCHUNK
    sh = [None] * NCHUNK
    for j in range(NBUF):
        gh[j] = gather(j)
    for j in range(NCHUNK):
        gh[j].wait()
        sh[j] = store(j)
        nj = j - 1 + NBUF
        if j >= 1 and nj < NCHUNK:
            sh[j - 1].wait()
            gh[nj] = gather(nj)
    for j in range(max(0, NCHUNK - NBUF), NCHUNK):
        sh[j].wait()


@jax.jit
def _lookup(flat_tokens, cu_seqlens, emb_table):
    mesh = plsc.VectorSubcoreMesh(core_axis_name="c", subcore_axis_name="s")
    nb = cu_seqlens.shape[0]
    run = pl.kernel(
        _lookup_kernel,
        mesh=mesh,
        out_type=(jax.ShapeDtypeStruct((TOTAL, EMB), jnp.float32),
                  jax.ShapeDtypeStruct((nb,), cu_seqlens.dtype)),
        scratch_types=(
            [pltpu.VMEM((TPW,), jnp.int32),
             pltpu.VMEM((nb,), jnp.int32)]
            + [pltpu.VMEM((CHUNK, EMB), jnp.float32) for _ in range(NBUF)]
            + [pltpu.SemaphoreType.DMA for _ in range(2 * NBUF + 1)]
        ),
    )
    return run(flat_tokens, cu_seqlens, emb_table)


def kernel(flat_tokens, cu_seqlens, emb_table):
    all_embs, boundaries = _lookup(flat_tokens, cu_seqlens, emb_table)
    return (all_embs, boundaries)


# speedup vs baseline: 1.0534x; 1.0252x over previous
import functools

import jax
import jax.numpy as jnp
from jax import lax
from jax.experimental import pallas as pl
from jax.experimental.pallas import tpu as pltpu
from jax.experimental.pallas import tpu_sc as plsc

VOCAB = 32000
EMB = 512
TOTAL = 16384

NC = 2
NS = 16
NW = NC * NS
TPW = TOTAL // NW
CHUNK = 32
NCHUNK = TPW // CHUNK
NBUF = 7


def _lookup_kernel(idx_hbm, cu_hbm, table_hbm, out_hbm, bnd_hbm,
                   idx_v, bnd_v, *scratch):
    rows = scratch[:NBUF]
    gsem = scratch[NBUF:2 * NBUF]
    ssem = scratch[2 * NBUF:3 * NBUF]
    isem = scratch[3 * NBUF]
    wid = lax.axis_index("s") * NC + lax.axis_index("c")
    base = wid * TPW
    ih = pltpu.async_copy(idx_hbm.at[pl.ds(base, TPW)], idx_v, isem)

    @pl.when(wid == 0)
    def _():
        pltpu.sync_copy(cu_hbm, bnd_v)
        pltpu.sync_copy(bnd_v, bnd_hbm)

    def gather(j):
        b = j % NBUF
        return pltpu.async_copy(
            table_hbm.at[idx_v.at[pl.ds(j * CHUNK, CHUNK)]], rows[b], gsem[b])

    def store(j):
        b = j % NBUF
        return pltpu.async_copy(
            rows[b], out_hbm.at[pl.ds(base + j * CHUNK, CHUNK)], ssem[b])

    ih.wait()
    gh = [None] * NCHUNK
    sh = [None] * NCHUNK
    for j in range(NBUF):
        gh[j] = gather(j)
    for j in range(NCHUNK):
        gh[j].wait()
        sh[j] = store(j)
        nj = j - 1 + NBUF
        if j >= 1 and nj < NCHUNK:
            sh[j - 1].wait()
            gh[nj] = gather(nj)
    for j in range(max(0, NCHUNK - NBUF), NCHUNK):
        sh[j].wait()


@jax.jit
def _lookup(flat_tokens, cu_seqlens, emb_table):
    mesh = plsc.VectorSubcoreMesh(core_axis_name="c", subcore_axis_name="s")
    nb = cu_seqlens.shape[0]
    run = pl.kernel(
        _lookup_kernel,
        mesh=mesh,
        out_type=(jax.ShapeDtypeStruct((TOTAL, EMB), jnp.float32),
                  jax.ShapeDtypeStruct((nb,), cu_seqlens.dtype)),
        scratch_types=(
            [pltpu.VMEM((TPW,), jnp.int32),
             pltpu.VMEM((nb,), jnp.int32)]
            + [pltpu.VMEM((CHUNK, EMB), jnp.float32) for _ in range(NBUF)]
            + [pltpu.SemaphoreType.DMA for _ in range(2 * NBUF + 1)]
        ),
    )
    return run(flat_tokens, cu_seqlens, emb_table)


def kernel(flat_tokens, cu_seqlens, emb_table):
    all_embs, boundaries = _lookup(flat_tokens, cu_seqlens, emb_table)
    return (all_embs, boundaries)


# speedup vs baseline: 1.0557x; 1.0022x over previous
import functools

import jax
import jax.numpy as jnp
from jax import lax
from jax.experimental import pallas as pl
from jax.experimental.pallas import tpu as pltpu
from jax.experimental.pallas import tpu_sc as plsc

VOCAB = 32000
EMB = 512
TOTAL = 16384

NC = 2
NS = 16
NW = NC * NS
TPW = TOTAL // NW
CHUNK = 16
NCHUNK = TPW // CHUNK
NBUF = 14


def _lookup_kernel(idx_hbm, cu_hbm, table_hbm, out_hbm, bnd_hbm,
                   idx_v, bnd_v, *scratch):
    rows = scratch[:NBUF]
    gsem = scratch[NBUF:2 * NBUF]
    ssem = scratch[2 * NBUF:3 * NBUF]
    isem = scratch[3 * NBUF]
    wid = lax.axis_index("s") * NC + lax.axis_index("c")
    base = wid * TPW
    ih = pltpu.async_copy(idx_hbm.at[pl.ds(base, TPW)], idx_v, isem)

    @pl.when(wid == 0)
    def _():
        pltpu.sync_copy(cu_hbm, bnd_v)
        pltpu.sync_copy(bnd_v, bnd_hbm)

    def gather(j):
        b = j % NBUF
        return pltpu.async_copy(
            table_hbm.at[idx_v.at[pl.ds(j * CHUNK, CHUNK)]], rows[b], gsem[b])

    def store(j):
        b = j % NBUF
        return pltpu.async_copy(
            rows[b], out_hbm.at[pl.ds(base + j * CHUNK, CHUNK)], ssem[b])

    ih.wait()
    gh = [None] * NCHUNK
    sh = [None] * NCHUNK
    for j in range(NBUF):
        gh[j] = gather(j)
    for j in range(NCHUNK):
        gh[j].wait()
        sh[j] = store(j)
        nj = j - 1 + NBUF
        if j >= 1 and nj < NCHUNK:
            sh[j - 1].wait()
            gh[nj] = gather(nj)
    for j in range(max(0, NCHUNK - NBUF), NCHUNK):
        sh[j].wait()


@jax.jit
def _lookup(flat_tokens, cu_seqlens, emb_table):
    mesh = plsc.VectorSubcoreMesh(core_axis_name="c", subcore_axis_name="s")
    nb = cu_seqlens.shape[0]
    run = pl.kernel(
        _lookup_kernel,
        mesh=mesh,
        out_type=(jax.ShapeDtypeStruct((TOTAL, EMB), jnp.float32),
                  jax.ShapeDtypeStruct((nb,), cu_seqlens.dtype)),
        scratch_types=(
            [pltpu.VMEM((TPW,), jnp.int32),
             pltpu.VMEM((nb,), jnp.int32)]
            + [pltpu.VMEM((CHUNK, EMB), jnp.float32) for _ in range(NBUF)]
            + [pltpu.SemaphoreType.DMA for _ in range(2 * NBUF + 1)]
        ),
    )
    return run(flat_tokens, cu_seqlens, emb_table)


def kernel(flat_tokens, cu_seqlens, emb_table):
    all_embs, boundaries = _lookup(flat_tokens, cu_seqlens, emb_table)
    return (all_embs, boundaries)


# speedup vs baseline: 1.0616x; 1.0055x over previous
import functools

import jax
import jax.numpy as jnp
from jax import lax
from jax.experimental import pallas as pl
from jax.experimental.pallas import tpu as pltpu
from jax.experimental.pallas import tpu_sc as plsc

VOCAB = 32000
EMB = 512
TOTAL = 16384

NC = 2
NS = 16
NW = NC * NS
TPW = TOTAL // NW
CHUNK = 32
NCHUNK = TPW // CHUNK
NBUF = 7


def _lookup_kernel(idx_hbm, cu_hbm, table_hbm, out_hbm, bnd_hbm,
                   idx_v, bnd_v, *scratch):
    rows = scratch[:NBUF]
    gsem = scratch[NBUF:2 * NBUF]
    ssem = scratch[2 * NBUF:3 * NBUF]
    isem = scratch[3 * NBUF]
    wid = lax.axis_index("s") * NC + lax.axis_index("c")
    base = wid * TPW
    ih = pltpu.async_copy(idx_hbm.at[pl.ds(base, TPW)], idx_v, isem)

    def gather(j):
        b = j % NBUF
        return pltpu.async_copy(
            table_hbm.at[idx_v.at[pl.ds(j * CHUNK, CHUNK)]], rows[b], gsem[b])

    def store(j):
        b = j % NBUF
        return pltpu.async_copy(
            rows[b], out_hbm.at[pl.ds(base + j * CHUNK, CHUNK)], ssem[b])

    ih.wait()
    gh = [None] * NCHUNK
    sh = [None] * NCHUNK
    for j in range(NBUF):
        gh[j] = gather(j)
    for j in range(NCHUNK):
        gh[j].wait()
        sh[j] = store(j)
        nj = j - 1 + NBUF
        if j >= 1 and nj < NCHUNK:
            sh[j - 1].wait()
            gh[nj] = gather(nj)
    @pl.when(wid == 0)
    def _():
        pltpu.sync_copy(cu_hbm, bnd_v)
        pltpu.sync_copy(bnd_v, bnd_hbm)
    for j in range(max(0, NCHUNK - NBUF), NCHUNK):
        sh[j].wait()


@jax.jit
def _lookup(flat_tokens, cu_seqlens, emb_table):
    mesh = plsc.VectorSubcoreMesh(core_axis_name="c", subcore_axis_name="s")
    nb = cu_seqlens.shape[0]
    run = pl.kernel(
        _lookup_kernel,
        mesh=mesh,
        out_type=(jax.ShapeDtypeStruct((TOTAL, EMB), jnp.float32),
                  jax.ShapeDtypeStruct((nb,), cu_seqlens.dtype)),
        scratch_types=(
            [pltpu.VMEM((TPW,), jnp.int32),
             pltpu.VMEM((nb,), jnp.int32)]
            + [pltpu.VMEM((CHUNK, EMB), jnp.float32) for _ in range(NBUF)]
            + [pltpu.SemaphoreType.DMA for _ in range(2 * NBUF + 1)]
        ),
    )
    return run(flat_tokens, cu_seqlens, emb_table)


def kernel(flat_tokens, cu_seqlens, emb_table):
    all_embs, boundaries = _lookup(flat_tokens, cu_seqlens, emb_table)
    return (all_embs, boundaries)
